# baseline (speedup 1.0000x reference)
import jax
import jax.numpy as jnp
from jax import lax
from jax.experimental import pallas as pl
from jax.experimental.pallas import tpu as pltpu
from jax.experimental.pallas import tpu_sc as plsc

B = 1024
T = 1024
EMB = 64

NC = 2
NS = 16
NB = 4
NT = 8
BBLK = B // NB
TBLK = T // NT
HALF = 128
NBUF = 4
AHEAD = 2


def _emb_body(idxt_hbm, table_hbm, pos_hbm, out_hbm, idx_v, pos_v, rows_v,
              *sems):
    gs = sems[:NBUF]
    ws = sems[NBUF:]
    c = lax.axis_index("c")
    s = lax.axis_index("s")
    wid = s * NC + c
    bb = wid // NT
    tb = wid % NT
    b0 = bb * BBLK
    t0 = tb * TBLK

    pltpu.sync_copy(pos_hbm.at[pl.ds(t0, TBLK)], pos_v)
    for h in range(2):
        pltpu.sync_copy(
            idxt_hbm.at[pl.ds(t0, TBLK), pl.ds(b0 + h * HALF, HALF)],
            idx_v.at[h])

    def gather(t, k):
        for h in range(2):
            pltpu.make_async_copy(table_hbm.at[idx_v.at[h, t]],
                                  rows_v.at[k, pl.ds(h * HALF, HALF)],
                                  gs[k]).start()

    def wait_gather(k):
        pltpu.make_async_copy(table_hbm.at[pl.ds(0, BBLK)], rows_v.at[k],
                              gs[k]).wait()

    def write(t, k):
        pltpu.make_async_copy(rows_v.at[k],
                              out_hbm.at[pl.ds(b0, BBLK), t0 + t],
                              ws[k]).start()

    def wait_write(k):
        pltpu.make_async_copy(rows_v.at[k],
                              out_hbm.at[pl.ds(b0, BBLK), t0],
                              ws[k]).wait()

    def add(t, k):
        pv = [pos_v[t, pl.ds(g * 16, 16)] for g in range(EMB // 16)]

        @pl.loop(0, BBLK, unroll=4)
        def _add(r):
            for g in range(EMB // 16):
                sl = pl.ds(g * 16, 16)
                rows_v[k, r, sl] = rows_v[k, r, sl] + pv[g]

    for t in range(AHEAD):
        gather(t, t)
    for t in range(2):
        wait_gather(t)
        add(t, t)
        write(t, t)
        gather(t + AHEAD, (t + AHEAD) % NBUF)

    NSTEADY = (TBLK - AHEAD - 2) // NBUF

    @pl.loop(0, NSTEADY)
    def _grp(q):
        for i in range(NBUF):
            t = q * NBUF + 2 + i
            k = (2 + i) % NBUF
            wait_gather(k)
            add(t, k)
            write(t, k)
            k2 = (k + AHEAD) % NBUF
            wait_write(k2)
            gather(t + AHEAD, k2)

    for t in range(TBLK - AHEAD, TBLK):
        k = t % NBUF
        wait_gather(k)
        add(t, k)
        write(t, k)
    for k in range(NBUF):
        wait_write(k)


@jax.jit
def _emb(input_ids_t, token_table, pos_table):
    mesh = plsc.VectorSubcoreMesh(core_axis_name="c", subcore_axis_name="s")
    f = pl.kernel(
        _emb_body,
        out_type=jax.ShapeDtypeStruct((B, T, EMB), jnp.float32),
        mesh=mesh,
        scratch_types=[
            pltpu.VMEM((2, TBLK, HALF), jnp.int32),
            pltpu.VMEM((TBLK, EMB), jnp.float32),
            pltpu.VMEM((NBUF, BBLK, EMB), jnp.float32),
        ] + [pltpu.SemaphoreType.DMA] * (2 * NBUF),
        compiler_params=pltpu.CompilerParams(use_tc_tiling_on_sc=False),
    )
    return f(input_ids_t, token_table, pos_table)


def kernel(input_ids, attn_mask, token_table, pos_table):
    idx_t = input_ids.astype(jnp.int32).T
    out = _emb(idx_t, token_table, pos_table)
    return (out, attn_mask)


# speedup vs baseline: 1.0005x; 1.0005x over previous
import jax
import jax.numpy as jnp
from jax import lax
from jax.experimental import pallas as pl
from jax.experimental.pallas import tpu as pltpu
from jax.experimental.pallas import tpu_sc as plsc

B = 1024
T = 1024
EMB = 64

NC = 2
NS = 16
NB = 8
NT = 4
BBLK = B // NB
TBLK = T // NT
NBUF = 8
AHEAD = NBUF - 2


def _emb_body(idxt_hbm, table_hbm, pos_hbm, out_hbm, idx_v, pos_v, rows_v,
              *sems):
    gs = sems[:NBUF]
    ws = sems[NBUF:]
    c = lax.axis_index("c")
    s = lax.axis_index("s")
    wid = s * NC + c
    bb = wid // NT
    tb = wid % NT
    b0 = bb * BBLK
    t0 = tb * TBLK

    pltpu.sync_copy(pos_hbm.at[pl.ds(t0, TBLK)], pos_v)
    pltpu.sync_copy(idxt_hbm.at[pl.ds(t0, TBLK), pl.ds(b0, BBLK)], idx_v)

    def gather(t, k):
        pltpu.make_async_copy(table_hbm.at[idx_v.at[t]], rows_v.at[k],
                              gs[k]).start()

    def wait_gather(k):
        pltpu.make_async_copy(table_hbm.at[idx_v.at[0]], rows_v.at[k],
                              gs[k]).wait()

    def write(t, k):
        pltpu.make_async_copy(rows_v.at[k],
                              out_hbm.at[pl.ds(b0, BBLK), t0 + t],
                              ws[k]).start()

    def wait_write(k):
        pltpu.make_async_copy(rows_v.at[k],
                              out_hbm.at[pl.ds(b0, BBLK), t0],
                              ws[k]).wait()

    def add(t, k):
        pv = [pos_v[t, pl.ds(g * 16, 16)] for g in range(EMB // 16)]

        @pl.loop(0, BBLK, unroll=4)
        def _add(r):
            for g in range(EMB // 16):
                sl = pl.ds(g * 16, 16)
                rows_v[k, r, sl] = rows_v[k, r, sl] + pv[g]

    for t in range(AHEAD):
        gather(t, t)
    for t in range(2):
        wait_gather(t)
        add(t, t)
        write(t, t)
        gather(t + AHEAD, (t + AHEAD) % NBUF)

    NSTEADY = (TBLK - AHEAD - 2) // NBUF

    @pl.loop(0, NSTEADY)
    def _grp(q):
        for i in range(NBUF):
            t = q * NBUF + 2 + i
            k = (2 + i) % NBUF
            wait_gather(k)
            add(t, k)
            write(t, k)
            k2 = (k + AHEAD) % NBUF
            wait_write(k2)
            gather(t + AHEAD, k2)

    for t in range(TBLK - AHEAD, TBLK):
        k = t % NBUF
        wait_gather(k)
        add(t, k)
        write(t, k)
    for k in range(NBUF):
        wait_write(k)


@jax.jit
def _emb(input_ids_t, token_table, pos_table):
    mesh = plsc.VectorSubcoreMesh(core_axis_name="c", subcore_axis_name="s")
    f = pl.kernel(
        _emb_body,
        out_type=jax.ShapeDtypeStruct((B, T, EMB), jnp.float32),
        mesh=mesh,
        scratch_types=[
            pltpu.VMEM((TBLK, BBLK), jnp.int32),
            pltpu.VMEM((TBLK, EMB), jnp.float32),
            pltpu.VMEM((NBUF, BBLK, EMB), jnp.float32),
        ] + [pltpu.SemaphoreType.DMA] * (2 * NBUF),
        compiler_params=pltpu.CompilerParams(use_tc_tiling_on_sc=False),
    )
    return f(input_ids_t, token_table, pos_table)


def kernel(input_ids, attn_mask, token_table, pos_table):
    idx_t = input_ids.astype(jnp.int32).T
    out = _emb(idx_t, token_table, pos_table)
    return (out, attn_mask)
